# Initial kernel scaffold; baseline (speedup 1.0000x reference)
import jax
import jax.numpy as jnp
from jax.experimental import pallas as pl


def kernel(x, pos_table):
    raise NotImplementedError("write your pallas kernel here")


# speedup vs baseline: 2.8465x; 2.8465x over previous
import jax
import jax.numpy as jnp
from jax.experimental import pallas as pl
from jax.experimental.pallas import tpu as pltpu

_BT = 512


def _body(x_ref, p_ref, o_ref):
    o_ref[...] = x_ref[...] + p_ref[...]


def kernel(x, pos_table):
    B, T, D = x.shape
    bt = min(_BT, T)
    return pl.pallas_call(
        _body,
        grid=(T // bt, B),
        in_specs=[
            pl.BlockSpec((1, bt, D), lambda t, b: (b, t, 0)),
            pl.BlockSpec((bt, D), lambda t, b: (t, 0)),
        ],
        out_specs=pl.BlockSpec((1, bt, D), lambda t, b: (b, t, 0)),
        out_shape=jax.ShapeDtypeStruct(x.shape, x.dtype),
    )(x, pos_table)


# speedup vs baseline: 3.1749x; 1.1154x over previous
import jax
import jax.numpy as jnp
from jax.experimental import pallas as pl
from jax.experimental.pallas import tpu as pltpu

_BT = 1024


def _body(x_ref, p_ref, o_ref):
    o_ref[...] = x_ref[...] + p_ref[...]


def kernel(x, pos_table):
    B, T, D = x.shape
    bt = min(_BT, T)
    return pl.pallas_call(
        _body,
        grid=(T // bt, B),
        in_specs=[
            pl.BlockSpec((1, bt, D), lambda t, b: (b, t, 0)),
            pl.BlockSpec((bt, D), lambda t, b: (t, 0)),
        ],
        out_specs=pl.BlockSpec((1, bt, D), lambda t, b: (b, t, 0)),
        out_shape=jax.ShapeDtypeStruct(x.shape, x.dtype),
    )(x, pos_table)


# speedup vs baseline: 3.3092x; 1.0423x over previous
import jax
import jax.numpy as jnp
from jax.experimental import pallas as pl
from jax.experimental.pallas import tpu as pltpu

_BT = 2048


def _body(x_ref, p_ref, o_ref):
    o_ref[...] = x_ref[...] + p_ref[...]


def kernel(x, pos_table):
    B, T, D = x.shape
    bt = min(_BT, T)
    return pl.pallas_call(
        _body,
        grid=(T // bt, B),
        in_specs=[
            pl.BlockSpec((1, bt, D), lambda t, b: (b, t, 0)),
            pl.BlockSpec((bt, D), lambda t, b: (t, 0)),
        ],
        out_specs=pl.BlockSpec((1, bt, D), lambda t, b: (b, t, 0)),
        out_shape=jax.ShapeDtypeStruct(x.shape, x.dtype),
    )(x, pos_table)
